# baseline (speedup 1.0000x reference)
import jax
import jax.numpy as jnp
from jax.experimental import pallas as pl
from jax.experimental.pallas import tpu as pltpu


def kernel(input, table):
    B, S, D = input.shape
    V = table.shape[0]
    F = S * D
    K = 8
    BBk = 32
    ROUNDS = B // (K * BBk)

    tbl2 = jnp.reshape(table, (1, V * D))

    def body(t_ref, out_ref, *rest):
        bufs = rest[:K]
        sems = rest[K:]
        emb = t_ref[:, :F]
        for k in range(K):
            bufs[k][...] = jnp.broadcast_to(emb, (BBk, F))
        for r in range(ROUNDS):
            for k in range(K):
                base = (r * K + k) * BBk
                pltpu.make_async_copy(
                    bufs[k], out_ref.at[pl.ds(base, BBk)], sems[k]).start()
        for r in range(ROUNDS):
            for k in range(K):
                base = (r * K + k) * BBk
                pltpu.make_async_copy(
                    bufs[k], out_ref.at[pl.ds(base, BBk)], sems[k]).wait()

    out2 = pl.pallas_call(
        body,
        in_specs=[pl.BlockSpec(memory_space=pltpu.MemorySpace.VMEM)],
        out_specs=pl.BlockSpec(memory_space=pl.ANY),
        out_shape=jax.ShapeDtypeStruct((B, F), jnp.float32),
        scratch_shapes=(
            [pltpu.VMEM((BBk, F), jnp.float32) for _ in range(K)]
            + [pltpu.SemaphoreType.DMA for _ in range(K)]
        ),
    )(tbl2)
    return jnp.reshape(out2, (B, S, D))
